# Initial kernel scaffold; baseline (speedup 1.0000x reference)
import jax
import jax.numpy as jnp
from jax.experimental import pallas as pl


def kernel(x_cmp, x_slk, x_trk, e_cs, e_cc, e_ct, e_tt, params):
    raise NotImplementedError("write your pallas kernel here")


# speedup vs baseline: 7.0209x; 7.0209x over previous
import functools

import jax
import jax.numpy as jnp
from jax import lax
from jax.experimental import pallas as pl
from jax.experimental.pallas import tpu as pltpu
from jax.experimental.pallas import tpu_sc as plsc

N = 50000
E = 800000
W64 = 64
HALF = 32
ACC_ROWS = 50048
ROWS_PER_TILE = ACC_ROWS // 16
EPT = 50176
E_PAD = EPT * 16
CHUNK = 128
BODIES = 98
CPB = 4
BLK = 2000
GRID_B = N // BLK

D_TOOL, D_DES, D_LAY, D_ROT, N_CONT = 7, 17, 5, 4, 8


def _tc_build_tables(x_cmp_ref, x_slk_ref, wc_ref, sc_ref, ones_ref,
                     ws_ref, ss_ref, csr_ref, tbl_ref, xr_ref):
    c = pl.program_id(0)
    xc = x_cmp_ref[...]
    lanes = lax.broadcasted_iota(jnp.int32, (BLK, 128), 1)
    t = xc[:, 0:1].astype(jnp.int32)
    d = xc[:, 1:2].astype(jnp.int32)
    y = xc[:, 2:3].astype(jnp.int32)
    r = xc[:, 3:4].astype(jnp.int32)
    oh = ((lanes == t) | (lanes == d + 16) | (lanes == y + 116)
          | (lanes == r + 124)).astype(jnp.float32)
    h64 = (jnp.dot(oh, wc_ref[...], preferred_element_type=jnp.float32)
           + jnp.dot(xc, sc_ref[...], preferred_element_type=jnp.float32)
           + ones_ref[...])
    tbl_ref[0, :, :] = jnp.where(c == 0, h64[:, :HALF], h64[:, HALF:])

    xs = x_slk_ref[...]
    ts = xs[:, 0:1].astype(jnp.int32)
    ds = xs[:, 1:2].astype(jnp.int32)
    oh_s = ((lanes == ts) | (lanes == ds + 16)).astype(jnp.float32)
    h_slk = (jnp.dot(oh_s, ws_ref[...], preferred_element_type=jnp.float32)
             + jnp.dot(xs, ss_ref[...], preferred_element_type=jnp.float32))
    xr_ref[...] = jnp.dot(h_slk, csr_ref[...], preferred_element_type=jnp.float32)


def _sc_segment_sum(tbl, srcv, dstv, zhbm, out,
                    acc, srcbuf, adjbuf, dstbuf, rows, gsem, ssem):
    c = lax.axis_index("c")
    s = lax.axis_index("s")
    r0 = s * ROWS_PER_TILE
    pltpu.sync_copy(zhbm.at[pl.ds(r0, ROWS_PER_TILE)],
                    acc.at[pl.ds(r0, ROWS_PER_TILE)])
    plsc.subcore_barrier()

    base_row = s * (EPT // CHUNK)
    off = c * N

    def body(b, carry):
        row0 = base_row + b * CPB
        pltpu.sync_copy(srcv.at[pl.ds(row0, CPB)], srcbuf)
        pltpu.sync_copy(dstv.at[pl.ds(row0, CPB)], dstbuf)
        for j in range(CPB):
            for g in range(8):
                adjbuf[j, pl.ds(g * 16, 16)] = (
                    srcbuf[j, pl.ds(g * 16, 16)] + off)
        gh = [pltpu.async_copy(tbl.at[adjbuf.at[j]], rows.at[j], gsem)
              for j in range(CPB)]
        for h in gh:
            h.wait()
        sh = [pltpu.async_copy(rows.at[j], acc.at[dstbuf.at[j]], ssem,
                               add=True)
              for j in range(CPB)]
        for h in sh:
            h.wait()
        return carry

    lax.fori_loop(0, BODIES, body, 0)
    plsc.subcore_barrier()
    pltpu.sync_copy(acc.at[pl.ds(r0, ROWS_PER_TILE)],
                    out.at[c, pl.ds(r0, ROWS_PER_TILE)])


def _tc_head(s0_ref, s1_ref, xr_ref, csl_ref, inw_ref, inb_ref,
             linw_ref, linb_ref, hw_ref, hb_ref, h_ref, heads_ref):
    s64 = jnp.concatenate([s0_ref[0], s1_ref[0]], axis=1)
    cnt = jnp.maximum(s64[:, 41:42], 1.0)
    mean = s64 / cnt
    o = jax.nn.relu(jnp.dot(mean, csl_ref[...],
                            preferred_element_type=jnp.float32) + xr_ref[...])
    h1 = jax.nn.relu(jnp.dot(o, inw_ref[...],
                             preferred_element_type=jnp.float32) + inb_ref[...])
    h2 = jax.nn.relu(jnp.dot(h1, linw_ref[...],
                             preferred_element_type=jnp.float32) + linb_ref[...])
    h_ref[...] = h2
    heads_ref[...] = jnp.dot(h2, hw_ref[...],
                             preferred_element_type=jnp.float32) + hb_ref[...]


def kernel(x_cmp, x_slk, x_trk, e_cs, e_cc, e_ct, e_tt, params):
    p = params
    f32 = jnp.float32

    wc = jnp.zeros((128, W64), f32)
    wc = wc.at[0:16, 0:7].set(p['tool_emb'])
    wc = wc.at[16:116, 7:24].set(p['des_emb'])
    wc = wc.at[116:124, 24:29].set(p['lay_emb'])
    wc = wc.at[124:128, 29:33].set(p['rot_emb'])
    sc = jnp.zeros((12, W64), f32)
    sc = sc.at[jnp.arange(4, 12), jnp.arange(33, 41)].set(1.0)
    ones_row = jnp.zeros((1, W64), f32).at[0, 41].set(1.0)

    ws = jnp.zeros((128, 32), f32)
    ws = ws.at[0:16, 0:7].set(p['tool_emb'])
    ws = ws.at[16:116, 7:24].set(p['des_emb'])
    ss = jnp.zeros((10, 32), f32)
    ss = ss.at[jnp.arange(2, 10), jnp.arange(24, 32)].set(1.0)

    csl = jnp.zeros((W64, 64), f32).at[0:41, :].set(p['cs_l'])
    hw = jnp.concatenate([p['rot_w'], p['x_w'], p['y_w']], axis=1)
    hb = jnp.concatenate([p['rot_b'], p['x_b'], p['y_b']])[None, :]

    src = jnp.concatenate([e_cs[0], jnp.zeros((E_PAD - E,), jnp.int32)])
    dst = jnp.concatenate([e_cs[1], jnp.full((E_PAD - E,), N, jnp.int32)])
    src2d = src.reshape(E_PAD // CHUNK, CHUNK)
    dst2d = dst.reshape(E_PAD // CHUNK, CHUNK)
    zhbm = jnp.zeros((ACC_ROWS, HALF), f32)

    tbl2, xr = pl.pallas_call(
        _tc_build_tables,
        grid=(2, GRID_B),
        in_specs=[
            pl.BlockSpec((BLK, 12), lambda c, b: (b, 0)),
            pl.BlockSpec((BLK, 10), lambda c, b: (b, 0)),
            pl.BlockSpec((128, W64), lambda c, b: (0, 0)),
            pl.BlockSpec((12, W64), lambda c, b: (0, 0)),
            pl.BlockSpec((1, W64), lambda c, b: (0, 0)),
            pl.BlockSpec((128, 32), lambda c, b: (0, 0)),
            pl.BlockSpec((10, 32), lambda c, b: (0, 0)),
            pl.BlockSpec((32, 64), lambda c, b: (0, 0)),
        ],
        out_specs=[
            pl.BlockSpec((1, BLK, HALF), lambda c, b: (c, b, 0)),
            pl.BlockSpec((BLK, 64), lambda c, b: (b, 0)),
        ],
        out_shape=[
            jax.ShapeDtypeStruct((2, N, HALF), f32),
            jax.ShapeDtypeStruct((N, 64), f32),
        ],
    )(x_cmp, x_slk, wc, sc, ones_row, ws, ss, p['cs_r'])
    tbl = tbl2.reshape(2 * N, HALF)

    sums = pl.kernel(
        _sc_segment_sum,
        out_type=jax.ShapeDtypeStruct((2, ACC_ROWS, HALF), f32),
        mesh=plsc.VectorSubcoreMesh(core_axis_name="c", subcore_axis_name="s"),
        compiler_params=pltpu.CompilerParams(use_tc_tiling_on_sc=False),
        scratch_types=[
            pltpu.VMEM_SHARED((ACC_ROWS, HALF), f32),
            pltpu.VMEM((CPB, CHUNK), jnp.int32),
            pltpu.VMEM((CPB, CHUNK), jnp.int32),
            pltpu.VMEM((CPB, CHUNK), jnp.int32),
            pltpu.VMEM((CPB, CHUNK, HALF), f32),
            pltpu.SemaphoreType.DMA,
            pltpu.SemaphoreType.DMA,
        ],
    )(tbl, src2d, dst2d, zhbm)

    h, heads = pl.pallas_call(
        _tc_head,
        grid=(GRID_B,),
        in_specs=[
            pl.BlockSpec((1, BLK, HALF), lambda b: (0, b, 0)),
            pl.BlockSpec((1, BLK, HALF), lambda b: (1, b, 0)),
            pl.BlockSpec((BLK, 64), lambda b: (b, 0)),
            pl.BlockSpec((W64, 64), lambda b: (0, 0)),
            pl.BlockSpec((64, 64), lambda b: (0, 0)),
            pl.BlockSpec((1, 64), lambda b: (0, 0)),
            pl.BlockSpec((64, 64), lambda b: (0, 0)),
            pl.BlockSpec((1, 64), lambda b: (0, 0)),
            pl.BlockSpec((64, 6), lambda b: (0, 0)),
            pl.BlockSpec((1, 6), lambda b: (0, 0)),
        ],
        out_specs=[
            pl.BlockSpec((BLK, 64), lambda b: (b, 0)),
            pl.BlockSpec((BLK, 6), lambda b: (b, 0)),
        ],
        out_shape=[
            jax.ShapeDtypeStruct((N, 64), f32),
            jax.ShapeDtypeStruct((N, 6), f32),
        ],
    )(sums, sums, xr, csl, p['in_w'], p['in_b'][None, :],
      p['lin_w'], p['lin_b'][None, :], hw, hb)

    rot_out = heads[:, 0:4]
    x_out = heads[:, 4:5]
    y_out = heads[:, 5:6]
    return (h, h, h, rot_out, x_out, y_out)


# speedup vs baseline: 8.6937x; 1.2382x over previous
import functools

import jax
import jax.numpy as jnp
from jax import lax
from jax.experimental import pallas as pl
from jax.experimental.pallas import tpu as pltpu
from jax.experimental.pallas import tpu_sc as plsc

N = 50000
E = 800000
W64 = 64
HALF = 32
ACC_ROWS = 50048
ROWS_PER_TILE = ACC_ROWS // 16
EPT = 50176
E_PAD = EPT * 16
CHUNK = 128
CPB = 2
SB_TOTAL = EPT // (CPB * CHUNK)
BLK = 2000
GRID_B = N // BLK

D_TOOL, D_DES, D_LAY, D_ROT, N_CONT = 7, 17, 5, 4, 8


def _tc_build_tables(x_cmp_ref, x_slk_ref, wc_ref, sc_ref, ones_ref,
                     ws_ref, ss_ref, csr_ref, tbl_ref, xr_ref):
    c = pl.program_id(0)
    xc = x_cmp_ref[...]
    lanes = lax.broadcasted_iota(jnp.int32, (BLK, 128), 1)
    t = xc[:, 0:1].astype(jnp.int32)
    d = xc[:, 1:2].astype(jnp.int32)
    y = xc[:, 2:3].astype(jnp.int32)
    r = xc[:, 3:4].astype(jnp.int32)
    oh = ((lanes == t) | (lanes == d + 16) | (lanes == y + 116)
          | (lanes == r + 124)).astype(jnp.float32)
    h64 = (jnp.dot(oh, wc_ref[...], preferred_element_type=jnp.float32)
           + jnp.dot(xc, sc_ref[...], preferred_element_type=jnp.float32)
           + ones_ref[...])
    tbl_ref[0, :, :] = jnp.where(c == 0, h64[:, :HALF], h64[:, HALF:])

    xs = x_slk_ref[...]
    ts = xs[:, 0:1].astype(jnp.int32)
    ds = xs[:, 1:2].astype(jnp.int32)
    oh_s = ((lanes == ts) | (lanes == ds + 16)).astype(jnp.float32)
    h_slk = (jnp.dot(oh_s, ws_ref[...], preferred_element_type=jnp.float32)
             + jnp.dot(xs, ss_ref[...], preferred_element_type=jnp.float32))
    xr_ref[...] = jnp.dot(h_slk, csr_ref[...], preferred_element_type=jnp.float32)


def _sc_segment_sum(tbl, srcv, dstv, zhbm, out,
                    acc, srcbuf, adjbuf, dstbuf, dstpend, rows,
                    isem, gsem, ssem):
    c = lax.axis_index("c")
    s = lax.axis_index("s")
    r0 = s * ROWS_PER_TILE
    pltpu.sync_copy(zhbm.at[pl.ds(r0, ROWS_PER_TILE)],
                    acc.at[pl.ds(r0, ROWS_PER_TILE)])
    plsc.subcore_barrier()

    base_row = s * (EPT // CHUNK)
    last_row = base_row + (SB_TOTAL - 1) * CPB
    off = c * N

    def fire_ids(sb, slot):
        row = jnp.minimum(base_row + sb * CPB, last_row)
        pltpu.async_copy(srcv.at[pl.ds(row, CPB)], srcbuf.at[slot], isem)
        pltpu.async_copy(dstv.at[pl.ds(row, CPB)], dstbuf.at[slot], isem)

    def wait_ids(slot):
        pltpu.make_async_copy(srcv.at[pl.ds(0, CPB)], srcbuf.at[slot],
                              isem).wait()
        pltpu.make_async_copy(dstv.at[pl.ds(0, CPB)], dstbuf.at[slot],
                              isem).wait()

    def compute_adj(b):
        for j in range(CPB):
            for g in range(8):
                adjbuf[b, j, pl.ds(g * 16, 16)] = (
                    srcbuf[b, j, pl.ds(g * 16, 16)] + off)

    def copy_pend(b):
        for j in range(CPB):
            for g in range(8):
                dstpend[b, j, pl.ds(g * 16, 16)] = (
                    dstbuf[b, j, pl.ds(g * 16, 16)])

    def fire_gathers(b):
        for j in range(CPB):
            pltpu.async_copy(tbl.at[adjbuf.at[b, j]], rows.at[b, j], gsem)

    def wait_gathers(b):
        for j in range(CPB):
            pltpu.make_async_copy(zhbm.at[pl.ds(0, CHUNK)], rows.at[b, j],
                                  gsem).wait()

    def fire_scatters(b):
        for j in range(CPB):
            pltpu.async_copy(rows.at[b, j], acc.at[dstpend.at[b, j]], ssem,
                             add=True)

    def wait_scatters(b):
        for j in range(CPB):
            pltpu.make_async_copy(zhbm.at[pl.ds(0, CHUNK)], rows.at[b, j],
                                  ssem).wait()

    fire_ids(0, 0)
    wait_ids(0)
    compute_adj(0)
    fire_ids(1, 1)
    fire_gathers(0)
    wait_ids(1)
    compute_adj(1)
    copy_pend(0)
    fire_ids(2, 0)
    fire_gathers(1)
    wait_gathers(0)
    fire_scatters(0)

    def body(i, carry):
        for b in (0, 1):
            sb = 2 * i + b
            ob = 1 - b
            wait_ids(b)
            compute_adj(b)
            copy_pend(ob)
            fire_ids(sb + 1, ob)
            wait_scatters(b)
            fire_gathers(b)
            wait_gathers(ob)
            fire_scatters(ob)
        return carry

    lax.fori_loop(1, SB_TOTAL // 2, body, 0)
    wait_ids(0)
    wait_scatters(0)
    copy_pend(1)
    wait_gathers(1)
    fire_scatters(1)
    wait_scatters(1)
    plsc.subcore_barrier()
    pltpu.sync_copy(acc.at[pl.ds(r0, ROWS_PER_TILE)],
                    out.at[c, pl.ds(r0, ROWS_PER_TILE)])


def _tc_head(s0_ref, s1_ref, xr_ref, csl_ref, inw_ref, inb_ref,
             linw_ref, linb_ref, hw_ref, hb_ref, h_ref, heads_ref):
    s64 = jnp.concatenate([s0_ref[0], s1_ref[0]], axis=1)
    cnt = jnp.maximum(s64[:, 41:42], 1.0)
    mean = s64 / cnt
    o = jax.nn.relu(jnp.dot(mean, csl_ref[...],
                            preferred_element_type=jnp.float32) + xr_ref[...])
    h1 = jax.nn.relu(jnp.dot(o, inw_ref[...],
                             preferred_element_type=jnp.float32) + inb_ref[...])
    h2 = jax.nn.relu(jnp.dot(h1, linw_ref[...],
                             preferred_element_type=jnp.float32) + linb_ref[...])
    h_ref[...] = h2
    heads_ref[...] = jnp.dot(h2, hw_ref[...],
                             preferred_element_type=jnp.float32) + hb_ref[...]


def kernel(x_cmp, x_slk, x_trk, e_cs, e_cc, e_ct, e_tt, params):
    p = params
    f32 = jnp.float32

    wc = jnp.zeros((128, W64), f32)
    wc = wc.at[0:16, 0:7].set(p['tool_emb'])
    wc = wc.at[16:116, 7:24].set(p['des_emb'])
    wc = wc.at[116:124, 24:29].set(p['lay_emb'])
    wc = wc.at[124:128, 29:33].set(p['rot_emb'])
    sc = jnp.zeros((12, W64), f32)
    sc = sc.at[jnp.arange(4, 12), jnp.arange(33, 41)].set(1.0)
    ones_row = jnp.zeros((1, W64), f32).at[0, 41].set(1.0)

    ws = jnp.zeros((128, 32), f32)
    ws = ws.at[0:16, 0:7].set(p['tool_emb'])
    ws = ws.at[16:116, 7:24].set(p['des_emb'])
    ss = jnp.zeros((10, 32), f32)
    ss = ss.at[jnp.arange(2, 10), jnp.arange(24, 32)].set(1.0)

    csl = jnp.zeros((W64, 64), f32).at[0:41, :].set(p['cs_l'])
    hw = jnp.concatenate([p['rot_w'], p['x_w'], p['y_w']], axis=1)
    hb = jnp.concatenate([p['rot_b'], p['x_b'], p['y_b']])[None, :]

    src = jnp.concatenate([e_cs[0], jnp.zeros((E_PAD - E,), jnp.int32)])
    dst = jnp.concatenate([e_cs[1], jnp.full((E_PAD - E,), N, jnp.int32)])
    src2d = src.reshape(E_PAD // CHUNK, CHUNK)
    dst2d = dst.reshape(E_PAD // CHUNK, CHUNK)
    zhbm = jnp.zeros((ACC_ROWS, HALF), f32)

    tbl2, xr = pl.pallas_call(
        _tc_build_tables,
        grid=(2, GRID_B),
        in_specs=[
            pl.BlockSpec((BLK, 12), lambda c, b: (b, 0)),
            pl.BlockSpec((BLK, 10), lambda c, b: (b, 0)),
            pl.BlockSpec((128, W64), lambda c, b: (0, 0)),
            pl.BlockSpec((12, W64), lambda c, b: (0, 0)),
            pl.BlockSpec((1, W64), lambda c, b: (0, 0)),
            pl.BlockSpec((128, 32), lambda c, b: (0, 0)),
            pl.BlockSpec((10, 32), lambda c, b: (0, 0)),
            pl.BlockSpec((32, 64), lambda c, b: (0, 0)),
        ],
        out_specs=[
            pl.BlockSpec((1, BLK, HALF), lambda c, b: (c, b, 0)),
            pl.BlockSpec((BLK, 64), lambda c, b: (b, 0)),
        ],
        out_shape=[
            jax.ShapeDtypeStruct((2, N, HALF), f32),
            jax.ShapeDtypeStruct((N, 64), f32),
        ],
    )(x_cmp, x_slk, wc, sc, ones_row, ws, ss, p['cs_r'])
    tbl = tbl2.reshape(2 * N, HALF)

    sums = pl.kernel(
        _sc_segment_sum,
        out_type=jax.ShapeDtypeStruct((2, ACC_ROWS, HALF), f32),
        mesh=plsc.VectorSubcoreMesh(core_axis_name="c", subcore_axis_name="s"),
        compiler_params=pltpu.CompilerParams(use_tc_tiling_on_sc=False),
        scratch_types=[
            pltpu.VMEM_SHARED((ACC_ROWS, HALF), f32),
            pltpu.VMEM((2, CPB, CHUNK), jnp.int32),
            pltpu.VMEM((2, CPB, CHUNK), jnp.int32),
            pltpu.VMEM((2, CPB, CHUNK), jnp.int32),
            pltpu.VMEM((2, CPB, CHUNK), jnp.int32),
            pltpu.VMEM((2, CPB, CHUNK, HALF), f32),
            pltpu.SemaphoreType.DMA,
            pltpu.SemaphoreType.DMA,
            pltpu.SemaphoreType.DMA,
        ],
    )(tbl, src2d, dst2d, zhbm)

    h, heads = pl.pallas_call(
        _tc_head,
        grid=(GRID_B,),
        in_specs=[
            pl.BlockSpec((1, BLK, HALF), lambda b: (0, b, 0)),
            pl.BlockSpec((1, BLK, HALF), lambda b: (1, b, 0)),
            pl.BlockSpec((BLK, 64), lambda b: (b, 0)),
            pl.BlockSpec((W64, 64), lambda b: (0, 0)),
            pl.BlockSpec((64, 64), lambda b: (0, 0)),
            pl.BlockSpec((1, 64), lambda b: (0, 0)),
            pl.BlockSpec((64, 64), lambda b: (0, 0)),
            pl.BlockSpec((1, 64), lambda b: (0, 0)),
            pl.BlockSpec((64, 6), lambda b: (0, 0)),
            pl.BlockSpec((1, 6), lambda b: (0, 0)),
        ],
        out_specs=[
            pl.BlockSpec((BLK, 64), lambda b: (b, 0)),
            pl.BlockSpec((BLK, 6), lambda b: (b, 0)),
        ],
        out_shape=[
            jax.ShapeDtypeStruct((N, 64), f32),
            jax.ShapeDtypeStruct((N, 6), f32),
        ],
    )(sums, sums, xr, csl, p['in_w'], p['in_b'][None, :],
      p['lin_w'], p['lin_b'][None, :], hw, hb)

    rot_out = heads[:, 0:4]
    x_out = heads[:, 4:5]
    y_out = heads[:, 5:6]
    return (h, h, h, rot_out, x_out, y_out)


# speedup vs baseline: 9.2693x; 1.0662x over previous
import functools

import jax
import jax.numpy as jnp
from jax import lax
from jax.experimental import pallas as pl
from jax.experimental.pallas import tpu as pltpu
from jax.experimental.pallas import tpu_sc as plsc

N = 50000
E = 800000
W64 = 64
HALF = 32
ACC_ROWS = 50048
ROWS_PER_TILE = ACC_ROWS // 16
EPT = 50176
E_PAD = EPT * 16
CHUNK = 128
CPB = 2
SB_TOTAL = EPT // (CPB * CHUNK)
BLK = 2000
GRID_B = N // BLK

D_TOOL, D_DES, D_LAY, D_ROT, N_CONT = 7, 17, 5, 4, 8


def _tc_build_tables(x_cmp_ref, x_slk_ref, wc_ref, sc_ref, ones_ref,
                     ws_ref, ss_ref, csr_ref, tbl_ref, xr_ref):
    xc = x_cmp_ref[...]
    lanes = lax.broadcasted_iota(jnp.int32, (BLK, 128), 1)
    t = xc[:, 0:1].astype(jnp.int32)
    d = xc[:, 1:2].astype(jnp.int32)
    y = xc[:, 2:3].astype(jnp.int32)
    r = xc[:, 3:4].astype(jnp.int32)
    oh = ((lanes == t) | (lanes == d + 16) | (lanes == y + 116)
          | (lanes == r + 124)).astype(jnp.float32)
    h64 = (jnp.dot(oh, wc_ref[...], preferred_element_type=jnp.float32)
           + jnp.dot(xc, sc_ref[...], preferred_element_type=jnp.float32)
           + ones_ref[...])
    tbl_ref[0, :, :] = h64[:, :HALF]
    tbl_ref[1, :, :] = h64[:, HALF:]

    xs = x_slk_ref[...]
    ts = xs[:, 0:1].astype(jnp.int32)
    ds = xs[:, 1:2].astype(jnp.int32)
    oh_s = ((lanes == ts) | (lanes == ds + 16)).astype(jnp.float32)
    h_slk = (jnp.dot(oh_s, ws_ref[...], preferred_element_type=jnp.float32)
             + jnp.dot(xs, ss_ref[...], preferred_element_type=jnp.float32))
    xr_ref[...] = jnp.dot(h_slk, csr_ref[...], preferred_element_type=jnp.float32)


def _sc_segment_sum(tbl, srcv, dstv, zhbm, out,
                    acc, srcbuf, adjbuf, dstbuf, dstpend, rows,
                    isem, gsem, ssem):
    c = lax.axis_index("c")
    s = lax.axis_index("s")
    r0 = s * ROWS_PER_TILE
    pltpu.sync_copy(zhbm.at[pl.ds(r0, ROWS_PER_TILE)],
                    acc.at[pl.ds(r0, ROWS_PER_TILE)])
    plsc.subcore_barrier()

    base_row = s * (EPT // CHUNK)
    last_row = base_row + (SB_TOTAL - 1) * CPB
    off = c * N

    def fire_ids(sb, slot):
        row = jnp.minimum(base_row + sb * CPB, last_row)
        pltpu.async_copy(srcv.at[pl.ds(row, CPB)], srcbuf.at[slot], isem)
        pltpu.async_copy(dstv.at[pl.ds(row, CPB)], dstbuf.at[slot], isem)

    def wait_ids(slot):
        pltpu.make_async_copy(srcv.at[pl.ds(0, CPB)], srcbuf.at[slot],
                              isem).wait()
        pltpu.make_async_copy(dstv.at[pl.ds(0, CPB)], dstbuf.at[slot],
                              isem).wait()

    def compute_adj(b):
        for j in range(CPB):
            for g in range(8):
                adjbuf[b, j, pl.ds(g * 16, 16)] = (
                    srcbuf[b, j, pl.ds(g * 16, 16)] + off)

    def copy_pend(b):
        for j in range(CPB):
            for g in range(8):
                dstpend[b, j, pl.ds(g * 16, 16)] = (
                    dstbuf[b, j, pl.ds(g * 16, 16)])

    def fire_gathers(b):
        for j in range(CPB):
            pltpu.async_copy(tbl.at[adjbuf.at[b, j]], rows.at[b, j], gsem)

    def wait_gathers(b):
        for j in range(CPB):
            pltpu.make_async_copy(zhbm.at[pl.ds(0, CHUNK)], rows.at[b, j],
                                  gsem).wait()

    def fire_scatters(b):
        for j in range(CPB):
            pltpu.async_copy(rows.at[b, j], acc.at[dstpend.at[b, j]], ssem,
                             add=True)

    def wait_scatters(b):
        for j in range(CPB):
            pltpu.make_async_copy(zhbm.at[pl.ds(0, CHUNK)], rows.at[b, j],
                                  ssem).wait()

    fire_ids(0, 0)
    wait_ids(0)
    compute_adj(0)
    fire_ids(1, 1)
    fire_gathers(0)
    wait_ids(1)
    compute_adj(1)
    copy_pend(0)
    fire_ids(2, 0)
    fire_gathers(1)
    wait_gathers(0)
    fire_scatters(0)

    def body(i, carry):
        for b in (0, 1):
            sb = 2 * i + b
            ob = 1 - b
            wait_ids(b)
            compute_adj(b)
            copy_pend(ob)
            fire_ids(sb + 1, ob)
            wait_scatters(b)
            fire_gathers(b)
            wait_gathers(ob)
            fire_scatters(ob)
        return carry

    lax.fori_loop(1, SB_TOTAL // 2, body, 0)
    wait_ids(0)
    wait_scatters(0)
    copy_pend(1)
    wait_gathers(1)
    fire_scatters(1)
    wait_scatters(1)
    plsc.subcore_barrier()
    pltpu.sync_copy(acc.at[pl.ds(r0, ROWS_PER_TILE)],
                    out.at[c, pl.ds(r0, ROWS_PER_TILE)])


def _tc_head(s0_ref, s1_ref, xr_ref, csl_ref, inw_ref, inb_ref,
             linw_ref, linb_ref, hw_ref, hb_ref,
             ha_ref, hb2_ref, hc_ref, rot_ref, x_ref, y_ref):
    s64 = jnp.concatenate([s0_ref[0], s1_ref[0]], axis=1)
    cnt = jnp.maximum(s64[:, 41:42], 1.0)
    mean = s64 / cnt
    o = jax.nn.relu(jnp.dot(mean, csl_ref[...],
                            preferred_element_type=jnp.float32) + xr_ref[...])
    h1 = jax.nn.relu(jnp.dot(o, inw_ref[...],
                             preferred_element_type=jnp.float32) + inb_ref[...])
    h2 = jax.nn.relu(jnp.dot(h1, linw_ref[...],
                             preferred_element_type=jnp.float32) + linb_ref[...])
    ha_ref[...] = h2
    hb2_ref[...] = h2
    hc_ref[...] = h2
    heads = jnp.dot(h2, hw_ref[...],
                    preferred_element_type=jnp.float32) + hb_ref[...]
    rot_ref[...] = heads[:, 0:4]
    x_ref[...] = heads[:, 4:5]
    y_ref[...] = heads[:, 5:6]


def kernel(x_cmp, x_slk, x_trk, e_cs, e_cc, e_ct, e_tt, params):
    p = params
    f32 = jnp.float32

    wc = jnp.zeros((128, W64), f32)
    wc = wc.at[0:16, 0:7].set(p['tool_emb'])
    wc = wc.at[16:116, 7:24].set(p['des_emb'])
    wc = wc.at[116:124, 24:29].set(p['lay_emb'])
    wc = wc.at[124:128, 29:33].set(p['rot_emb'])
    sc = jnp.zeros((12, W64), f32)
    sc = sc.at[jnp.arange(4, 12), jnp.arange(33, 41)].set(1.0)
    ones_row = jnp.zeros((1, W64), f32).at[0, 41].set(1.0)

    ws = jnp.zeros((128, 32), f32)
    ws = ws.at[0:16, 0:7].set(p['tool_emb'])
    ws = ws.at[16:116, 7:24].set(p['des_emb'])
    ss = jnp.zeros((10, 32), f32)
    ss = ss.at[jnp.arange(2, 10), jnp.arange(24, 32)].set(1.0)

    csl = jnp.zeros((W64, 64), f32).at[0:41, :].set(p['cs_l'])
    hw = jnp.concatenate([p['rot_w'], p['x_w'], p['y_w']], axis=1)
    hb = jnp.concatenate([p['rot_b'], p['x_b'], p['y_b']])[None, :]

    src = jnp.concatenate([e_cs[0], jnp.zeros((E_PAD - E,), jnp.int32)])
    dst = jnp.concatenate([e_cs[1], jnp.full((E_PAD - E,), N, jnp.int32)])
    src2d = src.reshape(E_PAD // CHUNK, CHUNK)
    dst2d = dst.reshape(E_PAD // CHUNK, CHUNK)
    zhbm = jnp.zeros((ACC_ROWS, HALF), f32)

    tbl2, xr = pl.pallas_call(
        _tc_build_tables,
        grid=(GRID_B,),
        in_specs=[
            pl.BlockSpec((BLK, 12), lambda b: (b, 0)),
            pl.BlockSpec((BLK, 10), lambda b: (b, 0)),
            pl.BlockSpec((128, W64), lambda b: (0, 0)),
            pl.BlockSpec((12, W64), lambda b: (0, 0)),
            pl.BlockSpec((1, W64), lambda b: (0, 0)),
            pl.BlockSpec((128, 32), lambda b: (0, 0)),
            pl.BlockSpec((10, 32), lambda b: (0, 0)),
            pl.BlockSpec((32, 64), lambda b: (0, 0)),
        ],
        out_specs=[
            pl.BlockSpec((2, BLK, HALF), lambda b: (0, b, 0)),
            pl.BlockSpec((BLK, 64), lambda b: (b, 0)),
        ],
        out_shape=[
            jax.ShapeDtypeStruct((2, N, HALF), f32),
            jax.ShapeDtypeStruct((N, 64), f32),
        ],
    )(x_cmp, x_slk, wc, sc, ones_row, ws, ss, p['cs_r'])
    tbl = tbl2.reshape(2 * N, HALF)

    sums = pl.kernel(
        _sc_segment_sum,
        out_type=jax.ShapeDtypeStruct((2, ACC_ROWS, HALF), f32),
        mesh=plsc.VectorSubcoreMesh(core_axis_name="c", subcore_axis_name="s"),
        compiler_params=pltpu.CompilerParams(use_tc_tiling_on_sc=False),
        scratch_types=[
            pltpu.VMEM_SHARED((ACC_ROWS, HALF), f32),
            pltpu.VMEM((2, CPB, CHUNK), jnp.int32),
            pltpu.VMEM((2, CPB, CHUNK), jnp.int32),
            pltpu.VMEM((2, CPB, CHUNK), jnp.int32),
            pltpu.VMEM((2, CPB, CHUNK), jnp.int32),
            pltpu.VMEM((2, CPB, CHUNK, HALF), f32),
            pltpu.SemaphoreType.DMA,
            pltpu.SemaphoreType.DMA,
            pltpu.SemaphoreType.DMA,
        ],
    )(tbl, src2d, dst2d, zhbm)

    ha, hb3, hc, rot, x, y = pl.pallas_call(
        _tc_head,
        grid=(GRID_B,),
        in_specs=[
            pl.BlockSpec((1, BLK, HALF), lambda b: (0, b, 0)),
            pl.BlockSpec((1, BLK, HALF), lambda b: (1, b, 0)),
            pl.BlockSpec((BLK, 64), lambda b: (b, 0)),
            pl.BlockSpec((W64, 64), lambda b: (0, 0)),
            pl.BlockSpec((64, 64), lambda b: (0, 0)),
            pl.BlockSpec((1, 64), lambda b: (0, 0)),
            pl.BlockSpec((64, 64), lambda b: (0, 0)),
            pl.BlockSpec((1, 64), lambda b: (0, 0)),
            pl.BlockSpec((64, 6), lambda b: (0, 0)),
            pl.BlockSpec((1, 6), lambda b: (0, 0)),
        ],
        out_specs=[
            pl.BlockSpec((BLK, 64), lambda b: (b, 0)),
            pl.BlockSpec((BLK, 64), lambda b: (b, 0)),
            pl.BlockSpec((BLK, 64), lambda b: (b, 0)),
            pl.BlockSpec((BLK, 4), lambda b: (b, 0)),
            pl.BlockSpec((BLK, 1), lambda b: (b, 0)),
            pl.BlockSpec((BLK, 1), lambda b: (b, 0)),
        ],
        out_shape=[
            jax.ShapeDtypeStruct((N, 64), f32),
            jax.ShapeDtypeStruct((N, 64), f32),
            jax.ShapeDtypeStruct((N, 64), f32),
            jax.ShapeDtypeStruct((N, 4), f32),
            jax.ShapeDtypeStruct((N, 1), f32),
            jax.ShapeDtypeStruct((N, 1), f32),
        ],
    )(sums, sums, xr, csl, p['in_w'], p['in_b'][None, :],
      p['lin_w'], p['lin_b'][None, :], hw, hb)
    return (ha, hb3, hc, rot, x, y)


# speedup vs baseline: 9.7022x; 1.0467x over previous
import functools

import jax
import jax.numpy as jnp
from jax import lax
from jax.experimental import pallas as pl
from jax.experimental.pallas import tpu as pltpu
from jax.experimental.pallas import tpu_sc as plsc

N = 50000
E = 800000
W64 = 48
HALF = 24
ACC_ROWS = 50048
ROWS_PER_TILE = ACC_ROWS // 16
EPT = 50176
E_PAD = EPT * 16
CHUNK = 128
CPB = 4
SB_TOTAL = EPT // (CPB * CHUNK)
BLK = 2000
GRID_B = N // BLK

D_TOOL, D_DES, D_LAY, D_ROT, N_CONT = 7, 17, 5, 4, 8


def _tc_build_tables(x_cmp_ref, x_slk_ref, wc_ref, sc_ref, ones_ref,
                     ws_ref, ss_ref, csr_ref, tbl_ref, xr_ref):
    xc = x_cmp_ref[...]
    lanes = lax.broadcasted_iota(jnp.int32, (BLK, 128), 1)
    t = xc[:, 0:1].astype(jnp.int32)
    d = xc[:, 1:2].astype(jnp.int32)
    y = xc[:, 2:3].astype(jnp.int32)
    r = xc[:, 3:4].astype(jnp.int32)
    oh = ((lanes == t) | (lanes == d + 16) | (lanes == y + 116)
          | (lanes == r + 124)).astype(jnp.float32)
    h64 = (jnp.dot(oh, wc_ref[...], preferred_element_type=jnp.float32)
           + jnp.dot(xc, sc_ref[...], preferred_element_type=jnp.float32)
           + ones_ref[...])
    tbl_ref[0, :, :] = h64[:, :HALF]
    tbl_ref[1, :, :] = h64[:, HALF:]

    xs = x_slk_ref[...]
    ts = xs[:, 0:1].astype(jnp.int32)
    ds = xs[:, 1:2].astype(jnp.int32)
    oh_s = ((lanes == ts) | (lanes == ds + 16)).astype(jnp.float32)
    h_slk = (jnp.dot(oh_s, ws_ref[...], preferred_element_type=jnp.float32)
             + jnp.dot(xs, ss_ref[...], preferred_element_type=jnp.float32))
    xr_ref[...] = jnp.dot(h_slk, csr_ref[...], preferred_element_type=jnp.float32)


def _sc_segment_sum(tbl, srcv, dstv, zhbm, out,
                    acc, srcbuf, adjbuf, dstbuf, dstpend, rows,
                    isem, gsem, ssem):
    c = lax.axis_index("c")
    s = lax.axis_index("s")
    r0 = s * ROWS_PER_TILE
    pltpu.sync_copy(zhbm.at[pl.ds(r0, ROWS_PER_TILE)],
                    acc.at[pl.ds(r0, ROWS_PER_TILE)])
    plsc.subcore_barrier()

    base_row = s * (EPT // CHUNK)
    last_row = base_row + (SB_TOTAL - 1) * CPB
    off = c * N

    def fire_ids(sb, slot):
        row = jnp.minimum(base_row + sb * CPB, last_row)
        pltpu.async_copy(srcv.at[pl.ds(row, CPB)], srcbuf.at[slot], isem)
        pltpu.async_copy(dstv.at[pl.ds(row, CPB)], dstbuf.at[slot], isem)

    def wait_ids(slot):
        pltpu.make_async_copy(srcv.at[pl.ds(0, CPB)], srcbuf.at[slot],
                              isem).wait()
        pltpu.make_async_copy(dstv.at[pl.ds(0, CPB)], dstbuf.at[slot],
                              isem).wait()

    def compute_adj(b):
        for j in range(CPB):
            for g in range(8):
                adjbuf[b, j, pl.ds(g * 16, 16)] = (
                    srcbuf[b, j, pl.ds(g * 16, 16)] + off)

    def copy_pend(b):
        for j in range(CPB):
            for g in range(8):
                dstpend[b, j, pl.ds(g * 16, 16)] = (
                    dstbuf[b, j, pl.ds(g * 16, 16)])

    def fire_gathers(b):
        for j in range(CPB):
            pltpu.async_copy(tbl.at[adjbuf.at[b, j]], rows.at[b, j], gsem)

    def wait_gathers(b):
        for j in range(CPB):
            pltpu.make_async_copy(zhbm.at[pl.ds(0, CHUNK)], rows.at[b, j],
                                  gsem).wait()

    def fire_scatters(b):
        for j in range(CPB):
            pltpu.async_copy(rows.at[b, j], acc.at[dstpend.at[b, j]], ssem,
                             add=True)

    def wait_scatters(b):
        for j in range(CPB):
            pltpu.make_async_copy(zhbm.at[pl.ds(0, CHUNK)], rows.at[b, j],
                                  ssem).wait()

    fire_ids(0, 0)
    wait_ids(0)
    compute_adj(0)
    fire_ids(1, 1)
    fire_gathers(0)
    wait_ids(1)
    compute_adj(1)
    copy_pend(0)
    fire_ids(2, 0)
    fire_gathers(1)
    wait_gathers(0)
    fire_scatters(0)

    def body(i, carry):
        for b in (0, 1):
            sb = 2 * i + b
            ob = 1 - b
            wait_ids(b)
            compute_adj(b)
            copy_pend(ob)
            fire_ids(sb + 1, ob)
            wait_scatters(b)
            fire_gathers(b)
            wait_gathers(ob)
            fire_scatters(ob)
        return carry

    lax.fori_loop(1, SB_TOTAL // 2, body, 0)
    wait_ids(0)
    wait_scatters(0)
    copy_pend(1)
    wait_gathers(1)
    fire_scatters(1)
    wait_scatters(1)
    plsc.subcore_barrier()
    pltpu.sync_copy(acc.at[pl.ds(r0, ROWS_PER_TILE)],
                    out.at[c, pl.ds(r0, ROWS_PER_TILE)])


def _tc_head(s0_ref, s1_ref, xr_ref, csl_ref, inw_ref, inb_ref,
             linw_ref, linb_ref, hw_ref, hb_ref,
             ha_ref, hb2_ref, hc_ref, rot_ref, x_ref, y_ref):
    s64 = jnp.concatenate([s0_ref[0], s1_ref[0]], axis=1)
    cnt = jnp.maximum(s64[:, 41:42], 1.0)
    mean = s64 / cnt
    o = jax.nn.relu(jnp.dot(mean, csl_ref[...],
                            preferred_element_type=jnp.float32) + xr_ref[...])
    h1 = jax.nn.relu(jnp.dot(o, inw_ref[...],
                             preferred_element_type=jnp.float32) + inb_ref[...])
    h2 = jax.nn.relu(jnp.dot(h1, linw_ref[...],
                             preferred_element_type=jnp.float32) + linb_ref[...])
    ha_ref[...] = h2
    hb2_ref[...] = h2
    hc_ref[...] = h2
    heads = jnp.dot(h2, hw_ref[...],
                    preferred_element_type=jnp.float32) + hb_ref[...]
    rot_ref[...] = heads[:, 0:4]
    x_ref[...] = heads[:, 4:5]
    y_ref[...] = heads[:, 5:6]


def kernel(x_cmp, x_slk, x_trk, e_cs, e_cc, e_ct, e_tt, params):
    p = params
    f32 = jnp.float32

    wc = jnp.zeros((128, W64), f32)
    wc = wc.at[0:16, 0:7].set(p['tool_emb'])
    wc = wc.at[16:116, 7:24].set(p['des_emb'])
    wc = wc.at[116:124, 24:29].set(p['lay_emb'])
    wc = wc.at[124:128, 29:33].set(p['rot_emb'])
    sc = jnp.zeros((12, W64), f32)
    sc = sc.at[jnp.arange(4, 12), jnp.arange(33, 41)].set(1.0)
    ones_row = jnp.zeros((1, W64), f32).at[0, 41].set(1.0)

    ws = jnp.zeros((128, 32), f32)
    ws = ws.at[0:16, 0:7].set(p['tool_emb'])
    ws = ws.at[16:116, 7:24].set(p['des_emb'])
    ss = jnp.zeros((10, 32), f32)
    ss = ss.at[jnp.arange(2, 10), jnp.arange(24, 32)].set(1.0)

    csl = jnp.zeros((W64, 64), f32).at[0:41, :].set(p['cs_l'])
    hw = jnp.concatenate([p['rot_w'], p['x_w'], p['y_w']], axis=1)
    hb = jnp.concatenate([p['rot_b'], p['x_b'], p['y_b']])[None, :]

    src = jnp.concatenate([e_cs[0], jnp.zeros((E_PAD - E,), jnp.int32)])
    dst = jnp.concatenate([e_cs[1], jnp.full((E_PAD - E,), N, jnp.int32)])
    src2d = src.reshape(E_PAD // CHUNK, CHUNK)
    dst2d = dst.reshape(E_PAD // CHUNK, CHUNK)
    zhbm = jnp.zeros((ACC_ROWS, HALF), f32)

    tbl2, xr = pl.pallas_call(
        _tc_build_tables,
        grid=(GRID_B,),
        in_specs=[
            pl.BlockSpec((BLK, 12), lambda b: (b, 0)),
            pl.BlockSpec((BLK, 10), lambda b: (b, 0)),
            pl.BlockSpec((128, W64), lambda b: (0, 0)),
            pl.BlockSpec((12, W64), lambda b: (0, 0)),
            pl.BlockSpec((1, W64), lambda b: (0, 0)),
            pl.BlockSpec((128, 32), lambda b: (0, 0)),
            pl.BlockSpec((10, 32), lambda b: (0, 0)),
            pl.BlockSpec((32, 64), lambda b: (0, 0)),
        ],
        out_specs=[
            pl.BlockSpec((2, BLK, HALF), lambda b: (0, b, 0)),
            pl.BlockSpec((BLK, 64), lambda b: (b, 0)),
        ],
        out_shape=[
            jax.ShapeDtypeStruct((2, N, HALF), f32),
            jax.ShapeDtypeStruct((N, 64), f32),
        ],
    )(x_cmp, x_slk, wc, sc, ones_row, ws, ss, p['cs_r'])
    tbl = tbl2.reshape(2 * N, HALF)

    sums = pl.kernel(
        _sc_segment_sum,
        out_type=jax.ShapeDtypeStruct((2, ACC_ROWS, HALF), f32),
        mesh=plsc.VectorSubcoreMesh(core_axis_name="c", subcore_axis_name="s"),
        compiler_params=pltpu.CompilerParams(use_tc_tiling_on_sc=False),
        scratch_types=[
            pltpu.VMEM_SHARED((ACC_ROWS, HALF), f32),
            pltpu.VMEM((2, CPB, CHUNK), jnp.int32),
            pltpu.VMEM((2, CPB, CHUNK), jnp.int32),
            pltpu.VMEM((2, CPB, CHUNK), jnp.int32),
            pltpu.VMEM((2, CPB, CHUNK), jnp.int32),
            pltpu.VMEM((2, CPB, CHUNK, HALF), f32),
            pltpu.SemaphoreType.DMA,
            pltpu.SemaphoreType.DMA,
            pltpu.SemaphoreType.DMA,
        ],
    )(tbl, src2d, dst2d, zhbm)

    ha, hb3, hc, rot, x, y = pl.pallas_call(
        _tc_head,
        grid=(GRID_B,),
        in_specs=[
            pl.BlockSpec((1, BLK, HALF), lambda b: (0, b, 0)),
            pl.BlockSpec((1, BLK, HALF), lambda b: (1, b, 0)),
            pl.BlockSpec((BLK, 64), lambda b: (b, 0)),
            pl.BlockSpec((W64, 64), lambda b: (0, 0)),
            pl.BlockSpec((64, 64), lambda b: (0, 0)),
            pl.BlockSpec((1, 64), lambda b: (0, 0)),
            pl.BlockSpec((64, 64), lambda b: (0, 0)),
            pl.BlockSpec((1, 64), lambda b: (0, 0)),
            pl.BlockSpec((64, 6), lambda b: (0, 0)),
            pl.BlockSpec((1, 6), lambda b: (0, 0)),
        ],
        out_specs=[
            pl.BlockSpec((BLK, 64), lambda b: (b, 0)),
            pl.BlockSpec((BLK, 64), lambda b: (b, 0)),
            pl.BlockSpec((BLK, 64), lambda b: (b, 0)),
            pl.BlockSpec((BLK, 4), lambda b: (b, 0)),
            pl.BlockSpec((BLK, 1), lambda b: (b, 0)),
            pl.BlockSpec((BLK, 1), lambda b: (b, 0)),
        ],
        out_shape=[
            jax.ShapeDtypeStruct((N, 64), f32),
            jax.ShapeDtypeStruct((N, 64), f32),
            jax.ShapeDtypeStruct((N, 64), f32),
            jax.ShapeDtypeStruct((N, 4), f32),
            jax.ShapeDtypeStruct((N, 1), f32),
            jax.ShapeDtypeStruct((N, 1), f32),
        ],
    )(sums, sums, xr, csl, p['in_w'], p['in_b'][None, :],
      p['lin_w'], p['lin_b'][None, :], hw, hb)
    return (ha, hb3, hc, rot, x, y)


# speedup vs baseline: 13.5881x; 1.4005x over previous
import functools

import jax
import jax.numpy as jnp
from jax import lax
from jax.experimental import pallas as pl
from jax.experimental.pallas import tpu as pltpu
from jax.experimental.pallas import tpu_sc as plsc

N = 50000
E = 800000
W64 = 48
HALF = 24
ACC_ROWS = 50048
ROWS_PER_TILE = ACC_ROWS // 16
EPT = 50176
E_PAD = EPT * 16
CHUNK = 128
CPB = 4
SB_TOTAL = EPT // (CPB * CHUNK)
BLK = 2048
GRID_B = (N + BLK - 1) // BLK

D_TOOL, D_DES, D_LAY, D_ROT, N_CONT = 7, 17, 5, 4, 8


def _tc_build_tables(x_cmp_ref, x_slk_ref, wc_ref, sc_ref, ones_ref,
                     ws_ref, ss_ref, csr_ref, tbl_ref, xr_ref):
    xc = x_cmp_ref[...]
    lanes = lax.broadcasted_iota(jnp.int32, (BLK, 128), 1)
    t = xc[:, 0:1].astype(jnp.int32)
    d = xc[:, 1:2].astype(jnp.int32)
    y = xc[:, 2:3].astype(jnp.int32)
    r = xc[:, 3:4].astype(jnp.int32)
    oh = ((lanes == t) | (lanes == d + 16) | (lanes == y + 116)
          | (lanes == r + 124)).astype(jnp.float32)
    h64 = (jnp.dot(oh, wc_ref[...], preferred_element_type=jnp.float32)
           + jnp.dot(xc, sc_ref[...], preferred_element_type=jnp.float32)
           + ones_ref[...])
    tbl_ref[0, :, :] = h64[:, :HALF]
    tbl_ref[1, :, :] = h64[:, HALF:]

    xs = x_slk_ref[...]
    ts = xs[:, 0:1].astype(jnp.int32)
    ds = xs[:, 1:2].astype(jnp.int32)
    oh_s = ((lanes == ts) | (lanes == ds + 16)).astype(jnp.float32)
    h_slk = (jnp.dot(oh_s, ws_ref[...], preferred_element_type=jnp.float32)
             + jnp.dot(xs, ss_ref[...], preferred_element_type=jnp.float32))
    xr_ref[...] = jnp.dot(csr_ref[...].T, h_slk.T,
                          preferred_element_type=jnp.float32)


def _sc_segment_sum(tbl, ids3, zhbm, out,
                    acc, srcbuf, adjbuf, dstbuf, dstpend, rows,
                    isem, gsem, ssem):
    c = lax.axis_index("c")
    s = lax.axis_index("s")
    r0 = s * ROWS_PER_TILE
    pltpu.sync_copy(zhbm.at[pl.ds(r0, ROWS_PER_TILE)],
                    acc.at[pl.ds(r0, ROWS_PER_TILE)])
    plsc.subcore_barrier()

    base_row = s * (EPT // CHUNK)
    last_row = base_row + (SB_TOTAL - 1) * CPB
    off = c * N

    def fire_ids(sb, slot):
        row = jnp.minimum(base_row + sb * CPB, last_row)
        pltpu.async_copy(ids3.at[0, pl.ds(row, CPB)], srcbuf.at[slot], isem)
        pltpu.async_copy(ids3.at[1, pl.ds(row, CPB)], dstbuf.at[slot], isem)

    def wait_ids(slot):
        pltpu.make_async_copy(ids3.at[0, pl.ds(0, CPB)], srcbuf.at[slot],
                              isem).wait()
        pltpu.make_async_copy(ids3.at[1, pl.ds(0, CPB)], dstbuf.at[slot],
                              isem).wait()

    def compute_adj(b):
        for j in range(CPB):
            for g in range(8):
                adjbuf[b, j, pl.ds(g * 16, 16)] = (
                    srcbuf[b, j, pl.ds(g * 16, 16)] + off)

    def copy_pend(b):
        for j in range(CPB):
            for g in range(8):
                dstpend[b, j, pl.ds(g * 16, 16)] = (
                    dstbuf[b, j, pl.ds(g * 16, 16)])

    def fire_gathers(b):
        for j in range(CPB):
            pltpu.async_copy(tbl.at[adjbuf.at[b, j]], rows.at[b, j], gsem)

    def wait_gathers(b):
        for j in range(CPB):
            pltpu.make_async_copy(zhbm.at[pl.ds(0, CHUNK)], rows.at[b, j],
                                  gsem).wait()

    def fire_scatters(b):
        for j in range(CPB):
            pltpu.async_copy(rows.at[b, j], acc.at[dstpend.at[b, j]], ssem,
                             add=True)

    def wait_scatters(b):
        for j in range(CPB):
            pltpu.make_async_copy(zhbm.at[pl.ds(0, CHUNK)], rows.at[b, j],
                                  ssem).wait()

    fire_ids(0, 0)
    wait_ids(0)
    compute_adj(0)
    fire_ids(1, 1)
    fire_gathers(0)
    wait_ids(1)
    compute_adj(1)
    copy_pend(0)
    fire_ids(2, 0)
    fire_gathers(1)
    wait_gathers(0)
    fire_scatters(0)

    def body(i, carry):
        for b in (0, 1):
            sb = 2 * i + b
            ob = 1 - b
            wait_ids(b)
            compute_adj(b)
            copy_pend(ob)
            fire_ids(sb + 1, ob)
            wait_scatters(b)
            fire_gathers(b)
            wait_gathers(ob)
            fire_scatters(ob)
        return carry

    lax.fori_loop(1, SB_TOTAL // 2, body, 0)
    wait_ids(0)
    wait_scatters(0)
    copy_pend(1)
    wait_gathers(1)
    fire_scatters(1)
    wait_scatters(1)
    plsc.subcore_barrier()
    pltpu.sync_copy(acc.at[pl.ds(r0, ROWS_PER_TILE)],
                    out.at[c, pl.ds(r0, ROWS_PER_TILE)])


def _tc_head(s0_ref, s1_ref, xr_ref, csl_ref, inw_ref, inb_ref,
             linw_ref, linb_ref, hw_ref, hb_ref,
             ha_ref, hb2_ref, hc_ref, rot_ref, x_ref, y_ref):
    s64t = jnp.concatenate([s0_ref[0], s1_ref[0]], axis=1).T
    cnt = jnp.maximum(s64t[41:42, :], 1.0)
    meant = s64t / cnt
    o = jax.nn.relu(jnp.dot(csl_ref[...].T, meant,
                            preferred_element_type=jnp.float32) + xr_ref[...])
    h1 = jax.nn.relu(jnp.dot(inw_ref[...].T, o,
                             preferred_element_type=jnp.float32) + inb_ref[...])
    h2 = jax.nn.relu(jnp.dot(linw_ref[...].T, h1,
                             preferred_element_type=jnp.float32) + linb_ref[...])
    ha_ref[...] = h2
    hb2_ref[...] = h2
    hc_ref[...] = h2
    heads = jnp.dot(hw_ref[...].T, h2,
                    preferred_element_type=jnp.float32) + hb_ref[...]
    rot_ref[...] = heads[0:4, :]
    x_ref[...] = heads[4:5, :]
    y_ref[...] = heads[5:6, :]


def kernel(x_cmp, x_slk, x_trk, e_cs, e_cc, e_ct, e_tt, params):
    p = params
    f32 = jnp.float32

    wc = jnp.zeros((128, W64), f32)
    wc = wc.at[0:16, 0:7].set(p['tool_emb'])
    wc = wc.at[16:116, 7:24].set(p['des_emb'])
    wc = wc.at[116:124, 24:29].set(p['lay_emb'])
    wc = wc.at[124:128, 29:33].set(p['rot_emb'])
    sc = jnp.zeros((12, W64), f32)
    sc = sc.at[jnp.arange(4, 12), jnp.arange(33, 41)].set(1.0)
    ones_row = jnp.zeros((1, W64), f32).at[0, 41].set(1.0)

    ws = jnp.zeros((128, 32), f32)
    ws = ws.at[0:16, 0:7].set(p['tool_emb'])
    ws = ws.at[16:116, 7:24].set(p['des_emb'])
    ss = jnp.zeros((10, 32), f32)
    ss = ss.at[jnp.arange(2, 10), jnp.arange(24, 32)].set(1.0)

    csl = jnp.zeros((W64, 64), f32).at[0:41, :].set(p['cs_l'])
    hw = jnp.concatenate([p['rot_w'], p['x_w'], p['y_w']], axis=1)
    hb = jnp.concatenate([p['rot_b'], p['x_b'], p['y_b']])[:, None]

    pad_ids = jnp.stack([jnp.zeros((E_PAD - E,), jnp.int32),
                         jnp.full((E_PAD - E,), N, jnp.int32)])
    ids3 = jnp.concatenate([e_cs, pad_ids], axis=1).reshape(
        2, E_PAD // CHUNK, CHUNK)
    zhbm = jnp.zeros((ACC_ROWS, HALF), f32)

    tbl2, xr = pl.pallas_call(
        _tc_build_tables,
        grid=(GRID_B,),
        in_specs=[
            pl.BlockSpec((BLK, 12), lambda b: (b, 0)),
            pl.BlockSpec((BLK, 10), lambda b: (b, 0)),
            pl.BlockSpec((128, W64), lambda b: (0, 0)),
            pl.BlockSpec((12, W64), lambda b: (0, 0)),
            pl.BlockSpec((1, W64), lambda b: (0, 0)),
            pl.BlockSpec((128, 32), lambda b: (0, 0)),
            pl.BlockSpec((10, 32), lambda b: (0, 0)),
            pl.BlockSpec((32, 64), lambda b: (0, 0)),
        ],
        out_specs=[
            pl.BlockSpec((2, BLK, HALF), lambda b: (0, b, 0)),
            pl.BlockSpec((64, BLK), lambda b: (0, b)),
        ],
        out_shape=[
            jax.ShapeDtypeStruct((2, N, HALF), f32),
            jax.ShapeDtypeStruct((64, N), f32),
        ],
    )(x_cmp, x_slk, wc, sc, ones_row, ws, ss, p['cs_r'])
    tbl = tbl2.reshape(2 * N, HALF)

    sums = pl.kernel(
        _sc_segment_sum,
        out_type=jax.ShapeDtypeStruct((2, ACC_ROWS, HALF), f32),
        mesh=plsc.VectorSubcoreMesh(core_axis_name="c", subcore_axis_name="s"),
        compiler_params=pltpu.CompilerParams(use_tc_tiling_on_sc=False),
        scratch_types=[
            pltpu.VMEM_SHARED((ACC_ROWS, HALF), f32),
            pltpu.VMEM((2, CPB, CHUNK), jnp.int32),
            pltpu.VMEM((2, CPB, CHUNK), jnp.int32),
            pltpu.VMEM((2, CPB, CHUNK), jnp.int32),
            pltpu.VMEM((2, CPB, CHUNK), jnp.int32),
            pltpu.VMEM((2, CPB, CHUNK, HALF), f32),
            pltpu.SemaphoreType.DMA,
            pltpu.SemaphoreType.DMA,
            pltpu.SemaphoreType.DMA,
        ],
    )(tbl, ids3, zhbm)

    ha, hb3, hc, rot, x, y = pl.pallas_call(
        _tc_head,
        grid=(GRID_B,),
        in_specs=[
            pl.BlockSpec((1, BLK, HALF), lambda b: (0, b, 0)),
            pl.BlockSpec((1, BLK, HALF), lambda b: (1, b, 0)),
            pl.BlockSpec((64, BLK), lambda b: (0, b)),
            pl.BlockSpec((W64, 64), lambda b: (0, 0)),
            pl.BlockSpec((64, 64), lambda b: (0, 0)),
            pl.BlockSpec((64, 1), lambda b: (0, 0)),
            pl.BlockSpec((64, 64), lambda b: (0, 0)),
            pl.BlockSpec((64, 1), lambda b: (0, 0)),
            pl.BlockSpec((64, 6), lambda b: (0, 0)),
            pl.BlockSpec((6, 1), lambda b: (0, 0)),
        ],
        out_specs=[
            pl.BlockSpec((64, BLK), lambda b: (0, b)),
            pl.BlockSpec((64, BLK), lambda b: (0, b)),
            pl.BlockSpec((64, BLK), lambda b: (0, b)),
            pl.BlockSpec((4, BLK), lambda b: (0, b)),
            pl.BlockSpec((1, BLK), lambda b: (0, b)),
            pl.BlockSpec((1, BLK), lambda b: (0, b)),
        ],
        out_shape=[
            jax.ShapeDtypeStruct((64, N), f32),
            jax.ShapeDtypeStruct((64, N), f32),
            jax.ShapeDtypeStruct((64, N), f32),
            jax.ShapeDtypeStruct((4, N), f32),
            jax.ShapeDtypeStruct((1, N), f32),
            jax.ShapeDtypeStruct((1, N), f32),
        ],
    )(sums, sums, xr, csl, p['in_w'], p['in_b'][:, None],
      p['lin_w'], p['lin_b'][:, None], hw, hb)
    return (ha.T, hb3.T, hc.T, rot.T, x.T, y.T)


# speedup vs baseline: 13.9641x; 1.0277x over previous
import functools

import jax
import jax.numpy as jnp
from jax import lax
from jax.experimental import pallas as pl
from jax.experimental.pallas import tpu as pltpu
from jax.experimental.pallas import tpu_sc as plsc

N = 50000
E = 800000
W64 = 48
HALF = 24
ACC_ROWS = 51200
ROWS_PER_TILE = ACC_ROWS // 16
EPT = 50176
E_PAD = EPT * 16
CHUNK = 128
CPB = 4
SB_TOTAL = EPT // (CPB * CHUNK)
BLK = 2048
GRID_B = (N + BLK - 1) // BLK

D_TOOL, D_DES, D_LAY, D_ROT, N_CONT = 7, 17, 5, 4, 8


def _tc_build_tables(x_cmp_ref, x_slk_ref, wc_ref, sc_ref, ones_ref,
                     ws_ref, ss_ref, csr_ref, tbl_ref, xr_ref):
    xc = x_cmp_ref[...]
    lanes = lax.broadcasted_iota(jnp.int32, (BLK, 128), 1)
    t = xc[:, 0:1].astype(jnp.int32)
    d = xc[:, 1:2].astype(jnp.int32)
    y = xc[:, 2:3].astype(jnp.int32)
    r = xc[:, 3:4].astype(jnp.int32)
    oh = ((lanes == t) | (lanes == d + 16) | (lanes == y + 116)
          | (lanes == r + 124)).astype(jnp.float32)
    h64 = (jnp.dot(oh, wc_ref[...], preferred_element_type=jnp.float32)
           + jnp.dot(xc, sc_ref[...], preferred_element_type=jnp.float32)
           + ones_ref[...])
    tbl_ref[...] = h64

    xs = x_slk_ref[...]
    ts = xs[:, 0:1].astype(jnp.int32)
    ds = xs[:, 1:2].astype(jnp.int32)
    oh_s = ((lanes == ts) | (lanes == ds + 16)).astype(jnp.float32)
    h_slk = (jnp.dot(oh_s, ws_ref[...], preferred_element_type=jnp.float32)
             + jnp.dot(xs, ss_ref[...], preferred_element_type=jnp.float32))
    xr_ref[...] = jnp.dot(csr_ref[...].T, h_slk.T,
                          preferred_element_type=jnp.float32)


def _sc_segment_sum(tbl, ids3, zhbm, out,
                    acc, srcbuf, adjbuf, dstbuf, dstpend, rows,
                    isem, gsem, ssem):
    c = lax.axis_index("c")
    s = lax.axis_index("s")
    r0 = s * ROWS_PER_TILE
    pltpu.sync_copy(zhbm.at[pl.ds(r0, ROWS_PER_TILE)],
                    acc.at[pl.ds(r0, ROWS_PER_TILE)])
    plsc.subcore_barrier()

    base_row = s * (EPT // CHUNK)
    last_row = base_row + (SB_TOTAL - 1) * CPB
    off = c

    def fire_ids(sb, slot):
        row = jnp.minimum(base_row + sb * CPB, last_row)
        pltpu.async_copy(ids3.at[0, pl.ds(row, CPB)], srcbuf.at[slot], isem)
        pltpu.async_copy(ids3.at[1, pl.ds(row, CPB)], dstbuf.at[slot], isem)

    def wait_ids(slot):
        pltpu.make_async_copy(ids3.at[0, pl.ds(0, CPB)], srcbuf.at[slot],
                              isem).wait()
        pltpu.make_async_copy(ids3.at[1, pl.ds(0, CPB)], dstbuf.at[slot],
                              isem).wait()

    def compute_adj(b):
        for j in range(CPB):
            for g in range(8):
                adjbuf[b, j, pl.ds(g * 16, 16)] = (
                    srcbuf[b, j, pl.ds(g * 16, 16)] * 2 + off)

    def copy_pend(b):
        for j in range(CPB):
            for g in range(8):
                dstpend[b, j, pl.ds(g * 16, 16)] = (
                    dstbuf[b, j, pl.ds(g * 16, 16)])

    def fire_gathers(b):
        for j in range(CPB):
            pltpu.async_copy(tbl.at[adjbuf.at[b, j]], rows.at[b, j], gsem)

    def wait_gathers(b):
        for j in range(CPB):
            pltpu.make_async_copy(zhbm.at[pl.ds(0, CHUNK)], rows.at[b, j],
                                  gsem).wait()

    def fire_scatters(b):
        for j in range(CPB):
            pltpu.async_copy(rows.at[b, j], acc.at[dstpend.at[b, j]], ssem,
                             add=True)

    def wait_scatters(b):
        for j in range(CPB):
            pltpu.make_async_copy(zhbm.at[pl.ds(0, CHUNK)], rows.at[b, j],
                                  ssem).wait()

    fire_ids(0, 0)
    wait_ids(0)
    compute_adj(0)
    fire_ids(1, 1)
    fire_gathers(0)
    wait_ids(1)
    compute_adj(1)
    copy_pend(0)
    fire_ids(2, 0)
    fire_gathers(1)
    wait_gathers(0)
    fire_scatters(0)

    def body(i, carry):
        for b in (0, 1):
            sb = 2 * i + b
            ob = 1 - b
            wait_ids(b)
            compute_adj(b)
            copy_pend(ob)
            fire_ids(sb + 1, ob)
            wait_scatters(b)
            fire_gathers(b)
            wait_gathers(ob)
            fire_scatters(ob)
        return carry

    lax.fori_loop(1, SB_TOTAL // 2, body, 0)
    wait_ids(0)
    wait_scatters(0)
    copy_pend(1)
    wait_gathers(1)
    fire_scatters(1)
    wait_scatters(1)
    plsc.subcore_barrier()
    pltpu.sync_copy(acc.at[pl.ds(r0, ROWS_PER_TILE)],
                    out.at[c, pl.ds(r0, ROWS_PER_TILE)])


def _tc_head(s0_ref, s1_ref, xr_ref, csl_ref, inw_ref, inb_ref,
             linw_ref, linb_ref, hw_ref, hb_ref,
             ha_ref, hb2_ref, hc_ref, rot_ref, x_ref, y_ref):
    s0 = s0_ref[0]
    s1 = s1_ref[0]
    s64t = jnp.concatenate([s0, s1], axis=1).T
    cnt = jnp.maximum(s64t[41:42, :], 1.0)
    meant = s64t / cnt
    o = jax.nn.relu(jnp.dot(csl_ref[...].T, meant,
                            preferred_element_type=jnp.float32) + xr_ref[...])
    h1 = jax.nn.relu(jnp.dot(inw_ref[...].T, o,
                             preferred_element_type=jnp.float32) + inb_ref[...])
    h2 = jax.nn.relu(jnp.dot(linw_ref[...].T, h1,
                             preferred_element_type=jnp.float32) + linb_ref[...])
    ha_ref[...] = h2
    hb2_ref[...] = h2
    hc_ref[...] = h2
    heads = jnp.dot(hw_ref[...].T, h2,
                    preferred_element_type=jnp.float32) + hb_ref[...]
    rot_ref[...] = heads[0:4, :]
    x_ref[...] = heads[4:5, :]
    y_ref[...] = heads[5:6, :]


def kernel(x_cmp, x_slk, x_trk, e_cs, e_cc, e_ct, e_tt, params):
    p = params
    f32 = jnp.float32

    wc = jnp.zeros((128, W64), f32)
    wc = wc.at[0:16, 0:7].set(p['tool_emb'])
    wc = wc.at[16:116, 7:24].set(p['des_emb'])
    wc = wc.at[116:124, 24:29].set(p['lay_emb'])
    wc = wc.at[124:128, 29:33].set(p['rot_emb'])
    sc = jnp.zeros((12, W64), f32)
    sc = sc.at[jnp.arange(4, 12), jnp.arange(33, 41)].set(1.0)
    ones_row = jnp.zeros((1, W64), f32).at[0, 41].set(1.0)

    ws = jnp.zeros((128, 32), f32)
    ws = ws.at[0:16, 0:7].set(p['tool_emb'])
    ws = ws.at[16:116, 7:24].set(p['des_emb'])
    ss = jnp.zeros((10, 32), f32)
    ss = ss.at[jnp.arange(2, 10), jnp.arange(24, 32)].set(1.0)

    csl = jnp.zeros((W64, 64), f32).at[0:41, :].set(p['cs_l'])
    hw = jnp.concatenate([p['rot_w'], p['x_w'], p['y_w']], axis=1)
    hb = jnp.concatenate([p['rot_b'], p['x_b'], p['y_b']])[:, None]

    pad_ids = jnp.stack([jnp.zeros((E_PAD - E,), jnp.int32),
                         jnp.full((E_PAD - E,), N, jnp.int32)])
    ids3 = jnp.concatenate([e_cs, pad_ids], axis=1).reshape(
        2, E_PAD // CHUNK, CHUNK)
    zhbm = jnp.zeros((ACC_ROWS, HALF), f32)

    tbl2, xr = pl.pallas_call(
        _tc_build_tables,
        grid=(GRID_B,),
        in_specs=[
            pl.BlockSpec((BLK, 12), lambda b: (b, 0)),
            pl.BlockSpec((BLK, 10), lambda b: (b, 0)),
            pl.BlockSpec((128, W64), lambda b: (0, 0)),
            pl.BlockSpec((12, W64), lambda b: (0, 0)),
            pl.BlockSpec((1, W64), lambda b: (0, 0)),
            pl.BlockSpec((128, 32), lambda b: (0, 0)),
            pl.BlockSpec((10, 32), lambda b: (0, 0)),
            pl.BlockSpec((32, 64), lambda b: (0, 0)),
        ],
        out_specs=[
            pl.BlockSpec((BLK, W64), lambda b: (b, 0)),
            pl.BlockSpec((64, BLK), lambda b: (0, b)),
        ],
        out_shape=[
            jax.ShapeDtypeStruct((N, W64), f32),
            jax.ShapeDtypeStruct((64, N), f32),
        ],
    )(x_cmp, x_slk, wc, sc, ones_row, ws, ss, p['cs_r'])
    tbl = tbl2.reshape(2 * N, HALF)

    sums = pl.kernel(
        _sc_segment_sum,
        out_type=jax.ShapeDtypeStruct((2, ACC_ROWS, HALF), f32),
        mesh=plsc.VectorSubcoreMesh(core_axis_name="c", subcore_axis_name="s"),
        compiler_params=pltpu.CompilerParams(use_tc_tiling_on_sc=False),
        scratch_types=[
            pltpu.VMEM_SHARED((ACC_ROWS, HALF), f32),
            pltpu.VMEM((2, CPB, CHUNK), jnp.int32),
            pltpu.VMEM((2, CPB, CHUNK), jnp.int32),
            pltpu.VMEM((2, CPB, CHUNK), jnp.int32),
            pltpu.VMEM((2, CPB, CHUNK), jnp.int32),
            pltpu.VMEM((2, CPB, CHUNK, HALF), f32),
            pltpu.SemaphoreType.DMA,
            pltpu.SemaphoreType.DMA,
            pltpu.SemaphoreType.DMA,
        ],
    )(tbl, ids3, zhbm)

    ha, hb3, hc, rot, x, y = pl.pallas_call(
        _tc_head,
        grid=(GRID_B,),
        in_specs=[
            pl.BlockSpec((1, BLK, HALF), lambda b: (0, b, 0)),
            pl.BlockSpec((1, BLK, HALF), lambda b: (1, b, 0)),
            pl.BlockSpec((64, BLK), lambda b: (0, b)),
            pl.BlockSpec((W64, 64), lambda b: (0, 0)),
            pl.BlockSpec((64, 64), lambda b: (0, 0)),
            pl.BlockSpec((64, 1), lambda b: (0, 0)),
            pl.BlockSpec((64, 64), lambda b: (0, 0)),
            pl.BlockSpec((64, 1), lambda b: (0, 0)),
            pl.BlockSpec((64, 6), lambda b: (0, 0)),
            pl.BlockSpec((6, 1), lambda b: (0, 0)),
        ],
        out_specs=[
            pl.BlockSpec((64, BLK), lambda b: (0, b)),
            pl.BlockSpec((64, BLK), lambda b: (0, b)),
            pl.BlockSpec((64, BLK), lambda b: (0, b)),
            pl.BlockSpec((4, BLK), lambda b: (0, b)),
            pl.BlockSpec((1, BLK), lambda b: (0, b)),
            pl.BlockSpec((1, BLK), lambda b: (0, b)),
        ],
        out_shape=[
            jax.ShapeDtypeStruct((64, N), f32),
            jax.ShapeDtypeStruct((64, N), f32),
            jax.ShapeDtypeStruct((64, N), f32),
            jax.ShapeDtypeStruct((4, N), f32),
            jax.ShapeDtypeStruct((1, N), f32),
            jax.ShapeDtypeStruct((1, N), f32),
        ],
    )(sums, sums, xr, csl, p['in_w'], p['in_b'][:, None],
      p['lin_w'], p['lin_b'][:, None], hw, hb)
    return (ha.T, hb3.T, hc.T, rot.T, x.T, y.T)


# speedup vs baseline: 14.8686x; 1.0648x over previous
import functools

import jax
import jax.numpy as jnp
from jax import lax
from jax.experimental import pallas as pl
from jax.experimental.pallas import tpu as pltpu
from jax.experimental.pallas import tpu_sc as plsc

N = 50000
E = 800000
W64 = 48
HALF = 24
ACC_ROWS = 51200
ROWS_PER_TILE = ACC_ROWS // 16
EPT = 50176
E_PAD = EPT * 16
CHUNK = 128
CPB = 4
SB_TOTAL = EPT // (CPB * CHUNK)
BLK = 2048
GRID_B = (N + BLK - 1) // BLK

D_TOOL, D_DES, D_LAY, D_ROT, N_CONT = 7, 17, 5, 4, 8


def _tc_build_tables(x_cmp_ref, wc_ref, sc_ref, ones_ref, tbl_ref):
    xc = x_cmp_ref[...]
    lanes = lax.broadcasted_iota(jnp.int32, (BLK, 128), 1)
    t = xc[:, 0:1].astype(jnp.int32)
    d = xc[:, 1:2].astype(jnp.int32)
    y = xc[:, 2:3].astype(jnp.int32)
    r = xc[:, 3:4].astype(jnp.int32)
    oh = ((lanes == t) | (lanes == d + 16) | (lanes == y + 116)
          | (lanes == r + 124)).astype(jnp.float32)
    h64 = (jnp.dot(oh, wc_ref[...], preferred_element_type=jnp.float32)
           + jnp.dot(xc, sc_ref[...], preferred_element_type=jnp.float32)
           + ones_ref[...])
    tbl_ref[...] = h64


def _tc_xr(x_slk_ref, ws_ref, ss_ref, csr_ref, xr_ref):
    xs = x_slk_ref[...]
    lanes = lax.broadcasted_iota(jnp.int32, (BLK, 128), 1)
    ts = xs[:, 0:1].astype(jnp.int32)
    ds = xs[:, 1:2].astype(jnp.int32)
    oh_s = ((lanes == ts) | (lanes == ds + 16)).astype(jnp.float32)
    h_slk = (jnp.dot(oh_s, ws_ref[...], preferred_element_type=jnp.float32)
             + jnp.dot(xs, ss_ref[...], preferred_element_type=jnp.float32))
    xr_ref[...] = jnp.dot(csr_ref[...].T, h_slk.T,
                          preferred_element_type=jnp.float32)


def _sc_segment_sum(tbl, ids3, zhbm, out,
                    acc, srcbuf, adjbuf, dstbuf, dstpend, rows,
                    isem, gsem, ssem):
    c = lax.axis_index("c")
    s = lax.axis_index("s")
    r0 = s * ROWS_PER_TILE
    pltpu.sync_copy(zhbm.at[pl.ds(r0, ROWS_PER_TILE)],
                    acc.at[pl.ds(r0, ROWS_PER_TILE)])
    plsc.subcore_barrier()

    base_row = s * (EPT // CHUNK)
    last_row = base_row + (SB_TOTAL - 1) * CPB
    off = c

    def fire_ids(sb, slot):
        row = jnp.minimum(base_row + sb * CPB, last_row)
        pltpu.async_copy(ids3.at[0, pl.ds(row, CPB)], srcbuf.at[slot], isem)
        pltpu.async_copy(ids3.at[1, pl.ds(row, CPB)], dstbuf.at[slot], isem)

    def wait_ids(slot):
        pltpu.make_async_copy(ids3.at[0, pl.ds(0, CPB)], srcbuf.at[slot],
                              isem).wait()
        pltpu.make_async_copy(ids3.at[1, pl.ds(0, CPB)], dstbuf.at[slot],
                              isem).wait()

    def compute_adj(b):
        for j in range(CPB):
            for g in range(8):
                adjbuf[b, j, pl.ds(g * 16, 16)] = (
                    srcbuf[b, j, pl.ds(g * 16, 16)] * 2 + off)

    def copy_pend(b):
        for j in range(CPB):
            for g in range(8):
                dstpend[b, j, pl.ds(g * 16, 16)] = (
                    dstbuf[b, j, pl.ds(g * 16, 16)])

    def fire_gathers(b):
        for j in range(CPB):
            pltpu.async_copy(tbl.at[adjbuf.at[b, j]], rows.at[b, j], gsem)

    def wait_gathers(b):
        for j in range(CPB):
            pltpu.make_async_copy(zhbm.at[pl.ds(0, CHUNK)], rows.at[b, j],
                                  gsem).wait()

    def fire_scatters(b):
        for j in range(CPB):
            pltpu.async_copy(rows.at[b, j], acc.at[dstpend.at[b, j]], ssem,
                             add=True)

    def wait_scatters(b):
        for j in range(CPB):
            pltpu.make_async_copy(zhbm.at[pl.ds(0, CHUNK)], rows.at[b, j],
                                  ssem).wait()

    fire_ids(0, 0)
    wait_ids(0)
    compute_adj(0)
    fire_ids(1, 1)
    fire_gathers(0)
    wait_ids(1)
    compute_adj(1)
    copy_pend(0)
    fire_ids(2, 0)
    fire_gathers(1)
    wait_gathers(0)
    fire_scatters(0)

    def body(i, carry):
        for b in (0, 1):
            sb = 2 * i + b
            ob = 1 - b
            wait_ids(b)
            compute_adj(b)
            copy_pend(ob)
            fire_ids(sb + 1, ob)
            wait_scatters(b)
            fire_gathers(b)
            wait_gathers(ob)
            fire_scatters(ob)
        return carry

    lax.fori_loop(1, SB_TOTAL // 2, body, 0)
    wait_ids(0)
    wait_scatters(0)
    copy_pend(1)
    wait_gathers(1)
    fire_scatters(1)
    wait_scatters(1)
    plsc.subcore_barrier()
    pltpu.sync_copy(acc.at[pl.ds(r0, ROWS_PER_TILE)],
                    out.at[c, pl.ds(r0, ROWS_PER_TILE)])


def _tc_head(s0_ref, s1_ref, xr_ref, csl_ref, inw_ref, inb_ref,
             linw_ref, linb_ref, hw_ref, hb_ref,
             ha_ref, hb2_ref, hc_ref, rot_ref, x_ref, y_ref):
    s0 = s0_ref[0]
    s1 = s1_ref[0]
    s64t = jnp.concatenate([s0, s1], axis=1).T
    cnt = jnp.maximum(s64t[41:42, :], 1.0)
    meant = s64t / cnt
    o = jax.nn.relu(jnp.dot(csl_ref[...].T, meant,
                            preferred_element_type=jnp.float32) + xr_ref[...])
    h1 = jax.nn.relu(jnp.dot(inw_ref[...].T, o,
                             preferred_element_type=jnp.float32) + inb_ref[...])
    h2 = jax.nn.relu(jnp.dot(linw_ref[...].T, h1,
                             preferred_element_type=jnp.float32) + linb_ref[...])
    ha_ref[...] = h2
    hb2_ref[...] = h2
    hc_ref[...] = h2
    heads = jnp.dot(hw_ref[...].T, h2,
                    preferred_element_type=jnp.float32) + hb_ref[...]
    rot_ref[...] = heads[0:4, :]
    x_ref[...] = heads[4:5, :]
    y_ref[...] = heads[5:6, :]


def kernel(x_cmp, x_slk, x_trk, e_cs, e_cc, e_ct, e_tt, params):
    p = params
    f32 = jnp.float32

    wc = jnp.zeros((128, W64), f32)
    wc = wc.at[0:16, 0:7].set(p['tool_emb'])
    wc = wc.at[16:116, 7:24].set(p['des_emb'])
    wc = wc.at[116:124, 24:29].set(p['lay_emb'])
    wc = wc.at[124:128, 29:33].set(p['rot_emb'])
    sc = jnp.zeros((12, W64), f32)
    sc = sc.at[jnp.arange(4, 12), jnp.arange(33, 41)].set(1.0)
    ones_row = jnp.zeros((1, W64), f32).at[0, 41].set(1.0)

    ws = jnp.zeros((128, 32), f32)
    ws = ws.at[0:16, 0:7].set(p['tool_emb'])
    ws = ws.at[16:116, 7:24].set(p['des_emb'])
    ss = jnp.zeros((10, 32), f32)
    ss = ss.at[jnp.arange(2, 10), jnp.arange(24, 32)].set(1.0)

    csl = jnp.zeros((W64, 64), f32).at[0:41, :].set(p['cs_l'])
    hw = jnp.concatenate([p['rot_w'], p['x_w'], p['y_w']], axis=1)
    hb = jnp.concatenate([p['rot_b'], p['x_b'], p['y_b']])[:, None]

    pad_ids = jnp.stack([jnp.zeros((E_PAD - E,), jnp.int32),
                         jnp.full((E_PAD - E,), N, jnp.int32)])
    ids3 = jnp.concatenate([e_cs, pad_ids], axis=1).reshape(
        2, E_PAD // CHUNK, CHUNK)
    zhbm = jnp.zeros((ACC_ROWS, HALF), f32)

    tbl2 = pl.pallas_call(
        _tc_build_tables,
        grid=(GRID_B,),
        in_specs=[
            pl.BlockSpec((BLK, 12), lambda b: (b, 0)),
            pl.BlockSpec((128, W64), lambda b: (0, 0)),
            pl.BlockSpec((12, W64), lambda b: (0, 0)),
            pl.BlockSpec((1, W64), lambda b: (0, 0)),
        ],
        out_specs=pl.BlockSpec((BLK, W64), lambda b: (b, 0)),
        out_shape=jax.ShapeDtypeStruct((N, W64), f32),
    )(x_cmp, wc, sc, ones_row)
    tbl = tbl2.reshape(2 * N, HALF)

    xr = pl.pallas_call(
        _tc_xr,
        grid=(GRID_B,),
        in_specs=[
            pl.BlockSpec((BLK, 10), lambda b: (b, 0)),
            pl.BlockSpec((128, 32), lambda b: (0, 0)),
            pl.BlockSpec((10, 32), lambda b: (0, 0)),
            pl.BlockSpec((32, 64), lambda b: (0, 0)),
        ],
        out_specs=pl.BlockSpec((64, BLK), lambda b: (0, b)),
        out_shape=jax.ShapeDtypeStruct((64, N), f32),
    )(x_slk, ws, ss, p['cs_r'])

    sums = pl.kernel(
        _sc_segment_sum,
        out_type=jax.ShapeDtypeStruct((2, ACC_ROWS, HALF), f32),
        mesh=plsc.VectorSubcoreMesh(core_axis_name="c", subcore_axis_name="s"),
        compiler_params=pltpu.CompilerParams(use_tc_tiling_on_sc=False),
        scratch_types=[
            pltpu.VMEM_SHARED((ACC_ROWS, HALF), f32),
            pltpu.VMEM((2, CPB, CHUNK), jnp.int32),
            pltpu.VMEM((2, CPB, CHUNK), jnp.int32),
            pltpu.VMEM((2, CPB, CHUNK), jnp.int32),
            pltpu.VMEM((2, CPB, CHUNK), jnp.int32),
            pltpu.VMEM((2, CPB, CHUNK, HALF), f32),
            pltpu.SemaphoreType.DMA,
            pltpu.SemaphoreType.DMA,
            pltpu.SemaphoreType.DMA,
        ],
    )(tbl, ids3, zhbm)

    ha, hb3, hc, rot, x, y = pl.pallas_call(
        _tc_head,
        grid=(GRID_B,),
        in_specs=[
            pl.BlockSpec((1, BLK, HALF), lambda b: (0, b, 0)),
            pl.BlockSpec((1, BLK, HALF), lambda b: (1, b, 0)),
            pl.BlockSpec((64, BLK), lambda b: (0, b)),
            pl.BlockSpec((W64, 64), lambda b: (0, 0)),
            pl.BlockSpec((64, 64), lambda b: (0, 0)),
            pl.BlockSpec((64, 1), lambda b: (0, 0)),
            pl.BlockSpec((64, 64), lambda b: (0, 0)),
            pl.BlockSpec((64, 1), lambda b: (0, 0)),
            pl.BlockSpec((64, 6), lambda b: (0, 0)),
            pl.BlockSpec((6, 1), lambda b: (0, 0)),
        ],
        out_specs=[
            pl.BlockSpec((64, BLK), lambda b: (0, b)),
            pl.BlockSpec((64, BLK), lambda b: (0, b)),
            pl.BlockSpec((64, BLK), lambda b: (0, b)),
            pl.BlockSpec((4, BLK), lambda b: (0, b)),
            pl.BlockSpec((1, BLK), lambda b: (0, b)),
            pl.BlockSpec((1, BLK), lambda b: (0, b)),
        ],
        out_shape=[
            jax.ShapeDtypeStruct((64, N), f32),
            jax.ShapeDtypeStruct((64, N), f32),
            jax.ShapeDtypeStruct((64, N), f32),
            jax.ShapeDtypeStruct((4, N), f32),
            jax.ShapeDtypeStruct((1, N), f32),
            jax.ShapeDtypeStruct((1, N), f32),
        ],
    )(sums, sums, xr, csl, p['in_w'], p['in_b'][:, None],
      p['lin_w'], p['lin_b'][:, None], hw, hb)
    return (ha.T, hb3.T, hc.T, rot.T, x.T, y.T)
